# baseline (speedup 1.0000x reference)
import functools

import jax
import jax.numpy as jnp
from jax import lax
from jax.experimental import pallas as pl
from jax.experimental.pallas import tpu as pltpu
from jax.experimental.pallas import tpu_sc as plsc

N = 32768
C = 81
D = 3

_NC = 2
_NS = 16
_L = 16
_NW = _NC * _NS
_BPW = N // _NW
_EPW = _BPW * D
_NGATHER = _EPW // 128

_mesh = plsc.VectorSubcoreMesh(core_axis_name="c", subcore_axis_name="s")


@functools.partial(
    pl.kernel,
    mesh=_mesh,
    out_type=jax.ShapeDtypeStruct((N * D,), jnp.float32),
    compiler_params=pltpu.CompilerParams(use_tc_tiling_on_sc=False),
    scratch_types=[
        pltpu.VMEM((_BPW,), jnp.int32),
        pltpu.VMEM((_NGATHER, 128), jnp.int32),
        pltpu.VMEM((_NGATHER, 128), jnp.int32),
        pltpu.VMEM((_EPW,), jnp.float32),
        pltpu.SemaphoreType.DMA,
        pltpu.SemaphoreType.DMA,
    ],
)
def _gather_elems(table_hbm, labels_hbm, out_hbm,
                  lab_v, gidx_v, sidx_v, vals_v, gsem, ssem):
    wid = lax.axis_index("s") * _NC + lax.axis_index("c")
    base = wid * _BPW
    pltpu.sync_copy(labels_hbm.at[pl.ds(base, _BPW)], lab_v)

    iota = lax.iota(jnp.int32, _L)
    for d in range(D):
        for j in range(_BPW // _L):
            lab16 = lab_v[pl.ds(j * _L, _L)]
            row16 = (base + j * _L) + iota
            p = d * _BPW + j * _L
            gidx_v[p // 128, pl.ds(p % 128, _L)] = row16 * (C * D) + lab16 * D + d
            sidx_v[p // 128, pl.ds(p % 128, _L)] = row16 * D + d

    gathers = [
        pltpu.async_copy(
            table_hbm.at[gidx_v.at[j]],
            vals_v.at[pl.ds(j * 128, 128)],
            gsem,
        )
        for j in range(_NGATHER)
    ]
    scatters = []
    for j in range(_NGATHER):
        gathers[j].wait()
        scatters.append(
            pltpu.async_copy(
                vals_v.at[pl.ds(j * 128, 128)],
                out_hbm.at[sidx_v.at[j]],
                ssem,
            )
        )
    for s in scatters:
        s.wait()


def kernel(depth_pred, labels):
    table = depth_pred.reshape(N * C * D)
    lab = labels.astype(jnp.int32)
    return _gather_elems(table, lab).reshape(N, D)
